# baseline (speedup 1.0000x reference)
import functools

import jax
import jax.numpy as jnp
from jax import lax
from jax.experimental import pallas as pl
from jax.experimental.pallas import tpu as pltpu
from jax.experimental.pallas import tpu_sc as plsc

NC = 2
NS = 16
L = 16
NW = NC * NS

B = 16384
D = 64
BPW = B // NW
CHUNK = 128
NCHUNK = BPW // CHUNK
GPC = CHUNK // L

_mesh = plsc.VectorSubcoreMesh(core_axis_name="c", subcore_axis_name="s")


@functools.partial(
    pl.kernel,
    out_type=jax.ShapeDtypeStruct((B,), jnp.float32),
    mesh=_mesh,
    scratch_types=[
        pltpu.VMEM((NCHUNK, CHUNK), jnp.int32),
        pltpu.VMEM((NCHUNK, CHUNK), jnp.int32),
        pltpu.VMEM((D, CHUNK), jnp.float32),
        pltpu.VMEM((D, CHUNK), jnp.float32),
        pltpu.VMEM((BPW,), jnp.float32),
        pltpu.SemaphoreType.DMA,
    ],
    compiler_params=pltpu.CompilerParams(
        needs_layout_passes=False, use_tc_tiling_on_sc=False),
)
def _svd_scores(users_hbm, items_hbm, uemb_hbm, iemb_hbm, out_hbm,
                uidx, iidx, ubuf, ibuf, out_v, sem):
    wid = lax.axis_index("s") * NC + lax.axis_index("c")
    base = wid * BPW

    for j in range(NCHUNK):
        pltpu.sync_copy(users_hbm.at[pl.ds(base + j * CHUNK, CHUNK)], uidx.at[j])
        pltpu.sync_copy(items_hbm.at[pl.ds(base + j * CHUNK, CHUNK)], iidx.at[j])

    for j in range(NCHUNK):
        def fire(d, carry, j=j):
            pltpu.async_copy(uemb_hbm.at[d].at[uidx.at[j]], ubuf.at[d], sem)
            pltpu.async_copy(iemb_hbm.at[d].at[iidx.at[j]], ibuf.at[d], sem)
            return carry
        lax.fori_loop(0, D, fire, 0)
        pltpu.make_async_copy(uemb_hbm.at[pl.ds(0, D), pl.ds(0, CHUNK)], ubuf, sem).wait()
        pltpu.make_async_copy(iemb_hbm.at[pl.ds(0, D), pl.ds(0, CHUNK)], ibuf, sem).wait()

        def accum(t, accs, j=j):
            d = t * 4
            new = []
            for g in range(GPC):
                a = accs[g]
                for dd in range(4):
                    sl = pl.ds(g * L, L)
                    a = a + ubuf[d + dd, sl] * ibuf[d + dd, sl]
                new.append(a)
            return tuple(new)

        accs = lax.fori_loop(
            0, D // 4, accum, tuple(jnp.zeros((L,), jnp.float32) for _ in range(GPC)))
        for g in range(GPC):
            out_v[pl.ds(j * CHUNK + g * L, L)] = accs[g]

    pltpu.sync_copy(out_v, out_hbm.at[pl.ds(base, BPW)])


def kernel(users, items, user_emb, item_emb):
    return _svd_scores(users, items, user_emb.T, item_emb.T)


# speedup vs baseline: 21.8964x; 21.8964x over previous
import functools

import jax
import jax.numpy as jnp
from jax import lax
from jax.experimental import pallas as pl
from jax.experimental.pallas import tpu as pltpu
from jax.experimental.pallas import tpu_sc as plsc

NC = 2
NS = 16
L = 16
NW = NC * NS

B = 16384
D = 64
W = 128
BPW = B // NW
GRP = BPW // L

_mesh = plsc.VectorSubcoreMesh(core_axis_name="c", subcore_axis_name="s")


@functools.partial(
    pl.kernel,
    out_type=jax.ShapeDtypeStruct((B,), jnp.float32),
    mesh=_mesh,
    scratch_types=[
        pltpu.VMEM((BPW,), jnp.int32),
        pltpu.VMEM((BPW,), jnp.int32),
        [pltpu.VMEM((2 * D, W), jnp.float32) for _ in range(2)],
        [pltpu.VMEM((2 * D, W), jnp.float32) for _ in range(2)],
        pltpu.VMEM((BPW,), jnp.float32),
        [pltpu.SemaphoreType.DMA for _ in range(2)],
    ],
    compiler_params=pltpu.CompilerParams(needs_layout_passes=False),
)
def _svd_scores(users_hbm, items_hbm, uemb_hbm, iemb_hbm, out_hbm,
                uidx, iidx, ustg, istg, out_v, sems):
    wid = lax.axis_index("s") * NC + lax.axis_index("c")
    base = wid * BPW

    pltpu.sync_copy(users_hbm.at[pl.ds(base, BPW)], uidx)
    pltpu.sync_copy(items_hbm.at[pl.ds(base, BPW)], iidx)

    lane = lax.iota(jnp.int32, L)

    def load_vecs(g):
        g = g & (GRP - 1)
        uv = uidx[pl.ds(g * L, L)]
        iv = iidx[pl.ds(g * L, L)]
        return (uv >> 7) << 7, uv & (W - 1), (iv >> 7) << 7, iv & (W - 1)

    def fire(vecs, e8, par):
        ucol, _, icol, _ = vecs
        for half in range(2):
            e = 2 * e8 + half
            pltpu.async_copy(
                uemb_hbm.at[pl.ds(0, D), pl.ds(pl.multiple_of(ucol[e], W), W)],
                ustg[par].at[pl.ds(half * D, D)], sems[par])
            pltpu.async_copy(
                iemb_hbm.at[pl.ds(0, D), pl.ds(pl.multiple_of(icol[e], W), W)],
                istg[par].at[pl.ds(half * D, D)], sems[par])

    def drain(par):
        dummy = uemb_hbm.at[pl.ds(0, D), pl.ds(0, W)]
        for buf in (ustg[par], istg[par]):
            pltpu.make_async_copy(dummy, buf.at[pl.ds(0, D)], sems[par]).wait()
            pltpu.make_async_copy(dummy, buf.at[pl.ds(D, D)], sems[par]).wait()

    def extract(vecs, e8, par, merged):
        _, uloc, _, iloc = vecs
        for half in range(2):
            e = 2 * e8 + half
            ucols = jnp.full((L,), 0, jnp.int32) + uloc[e]
            icols = jnp.full((L,), 0, jnp.int32) + iloc[e]
            acc = None
            for k in range(D // L):
                rows = half * D + k * L + lane
                p = plsc.load_gather(ustg[par], [rows, ucols]) * \
                    plsc.load_gather(istg[par], [rows, icols])
                acc = p if acc is None else acc + p
            merged = jnp.where(lane == e, jnp.sum(acc), merged)
        return merged

    vecs0 = load_vecs(0)
    fire(vecs0, 0, 0)

    def group_body(g, vecs):
        nvecs = load_vecs(g + 1)
        merged = jnp.zeros((L,), jnp.float32)
        for e8 in range(8):
            if e8 < 7:
                fire(vecs, e8 + 1, (e8 + 1) & 1)
            else:
                @pl.when(g < GRP - 1)
                def _(nvecs=nvecs):
                    fire(nvecs, 0, 0)
            drain(e8 & 1)
            merged = extract(vecs, e8, e8 & 1, merged)
        out_v[pl.ds(g * L, L)] = merged
        return nvecs

    lax.fori_loop(0, GRP, group_body, vecs0)

    pltpu.sync_copy(out_v, out_hbm.at[pl.ds(base, BPW)])


def kernel(users, items, user_emb, item_emb):
    return _svd_scores(users, items, user_emb.T, item_emb.T)
